# baseline (speedup 1.0000x reference)
import functools

import jax
import jax.numpy as jnp
from jax import lax
from jax.experimental import pallas as pl
from jax.experimental.pallas import tpu as pltpu
from jax.experimental.pallas import tpu_sc as plsc

_GATHER_DNUMS = lax.GatherDimensionNumbers(
    offset_dims=(), collapsed_slice_dims=(0,), start_index_map=(0,))


def _shuffle(v, perm):
    return lax.gather(v, perm[:, None], _GATHER_DNUMS, slice_sizes=(1,),
                      mode=lax.GatherScatterMode.PROMISE_IN_BOUNDS)


L = 16
NC = 2
NS = 16
NW = NC * NS
CHUNK = 128


def _make_kernel(B, D, V):
    n_chunks = B // (NW * CHUNK)
    b_per_w = n_chunks * CHUNK
    d_vecs = D // L

    mesh = plsc.VectorSubcoreMesh(core_axis_name="c", subcore_axis_name="s")

    @functools.partial(
        pl.kernel,
        mesh=mesh,
        out_type=jax.ShapeDtypeStruct((B,), jnp.float32),
        scratch_types=[
            pltpu.VMEM((CHUNK,), jnp.int32),
            pltpu.VMEM((CHUNK,), jnp.int32),
            pltpu.VMEM((CHUNK, D), jnp.float32),
            pltpu.VMEM((CHUNK, D), jnp.float32),
            pltpu.VMEM((CHUNK, D), jnp.float32),
            pltpu.VMEM((b_per_w,), jnp.float32),
            pltpu.SemaphoreType.DMA,
            pltpu.SemaphoreType.DMA,
            pltpu.SemaphoreType.DMA,
        ],
    )
    def triplet_kernel(mention_hbm, h_hbm, t_hbm, table_hbm, out_hbm,
                       hidx_v, tidx_v, hrows, trows, mrows, outv,
                       hsem, tsem, msem):
        wid = lax.axis_index("s") * NC + lax.axis_index("c")
        iot = lax.iota(jnp.int32, L)

        for c in range(n_chunks):
            blk = wid * n_chunks + c
            pltpu.sync_copy(h_hbm.at[blk], hidx_v)
            pltpu.sync_copy(t_hbm.at[blk], tidx_v)
            hcp = pltpu.async_copy(table_hbm.at[hidx_v], hrows, hsem)
            tcp = pltpu.async_copy(table_hbm.at[tidx_v], trows, tsem)
            mcp = pltpu.async_copy(mention_hbm.at[blk], mrows, msem)
            hcp.wait()
            tcp.wait()
            mcp.wait()

            def group_body(g, carry):
                tot = jnp.zeros((L,), jnp.float32)
                for r in range(L):
                    row = g * L + r
                    acc = jnp.zeros((L,), jnp.float32)
                    for j in range(d_vecs):
                        hv = hrows[row, pl.ds(j * L, L)]
                        tv = trows[row, pl.ds(j * L, L)]
                        mv = mrows[row, pl.ds(j * L, L)]
                        d = (hv + mv) - tv
                        acc = acc + d * d
                    for sh in (8, 4, 2, 1):
                        acc = acc + _shuffle(acc, iot ^ sh)
                    tot = jnp.where(iot == r, acc, tot)
                outv[pl.ds(c * CHUNK + g * L, L)] = tot
                return carry

            lax.fori_loop(0, CHUNK // L, group_body, 0)

        pltpu.sync_copy(outv, out_hbm.at[pl.ds(wid * b_per_w, b_per_w)])

    return triplet_kernel


def _finish_kernel(B):
    def body(ss_ref, out_ref):
        out_ref[...] = -jnp.sqrt(ss_ref[...])

    return pl.pallas_call(
        body, out_shape=jax.ShapeDtypeStruct((B // 128, 128), jnp.float32))


def kernel(mention, h, t, emb_table):
    B, D = mention.shape
    V = emb_table.shape[0]
    n_blocks = B // CHUNK
    h2 = h.reshape(n_blocks, CHUNK)
    t2 = t.reshape(n_blocks, CHUNK)
    mention3 = mention.reshape(n_blocks, CHUNK, D)
    ss = _make_kernel(B, D, V)(mention3, h2, t2, emb_table)
    return _finish_kernel(B)(ss.reshape(B // 128, 128)).reshape(B)


# speedup vs baseline: 1.2028x; 1.2028x over previous
import functools

import jax
import jax.numpy as jnp
from jax import lax
from jax.experimental import pallas as pl
from jax.experimental.pallas import tpu as pltpu
from jax.experimental.pallas import tpu_sc as plsc

_GATHER_DNUMS = lax.GatherDimensionNumbers(
    offset_dims=(), collapsed_slice_dims=(0,), start_index_map=(0,))


def _shuffle(v, perm):
    return lax.gather(v, perm[:, None], _GATHER_DNUMS, slice_sizes=(1,),
                      mode=lax.GatherScatterMode.PROMISE_IN_BOUNDS)


L = 16
NC = 2
NS = 16
NW = NC * NS
CHUNK = 128


def _make_kernel(B, D, V):
    n_chunks = B // (NW * CHUNK)
    b_per_w = n_chunks * CHUNK
    d_vecs = D // L

    mesh = plsc.VectorSubcoreMesh(core_axis_name="c", subcore_axis_name="s")

    @functools.partial(
        pl.kernel,
        mesh=mesh,
        out_type=jax.ShapeDtypeStruct((B,), jnp.float32),
        scratch_types=[
            pltpu.VMEM((b_per_w,), jnp.int32),
            pltpu.VMEM((b_per_w,), jnp.int32),
            pltpu.VMEM((2, CHUNK, D), jnp.float32),
            pltpu.VMEM((2, CHUNK, D), jnp.float32),
            pltpu.VMEM((2, CHUNK, D), jnp.float32),
            pltpu.VMEM((b_per_w,), jnp.float32),
            pltpu.SemaphoreType.DMA,
            pltpu.SemaphoreType.DMA,
            pltpu.SemaphoreType.DMA,
            pltpu.SemaphoreType.DMA,
            pltpu.SemaphoreType.DMA,
            pltpu.SemaphoreType.DMA,
        ],
    )
    def triplet_kernel(mention_hbm, h_hbm, t_hbm, table_hbm, out_hbm,
                       hidx, tidx, hrows, trows, mrows, outv,
                       hs0, hs1, ts0, ts1, ms0, ms1):
        wid = lax.axis_index("s") * NC + lax.axis_index("c")
        iot = lax.iota(jnp.int32, L)
        base = wid * b_per_w
        hsem = (hs0, hs1)
        tsem = (ts0, ts1)
        msem = (ms0, ms1)

        pltpu.sync_copy(h_hbm.at[pl.ds(base, b_per_w)], hidx)
        pltpu.sync_copy(t_hbm.at[pl.ds(base, b_per_w)], tidx)

        def fire(c):
            p = c & 1
            return (
                pltpu.async_copy(
                    table_hbm.at[hidx.at[pl.ds(c * CHUNK, CHUNK)]],
                    hrows.at[p], hsem[p]),
                pltpu.async_copy(
                    table_hbm.at[tidx.at[pl.ds(c * CHUNK, CHUNK)]],
                    trows.at[p], tsem[p]),
                pltpu.async_copy(
                    mention_hbm.at[pl.ds(base + c * CHUNK, CHUNK)],
                    mrows.at[p], msem[p]),
            )

        def compute(c):
            p = c & 1

            def group_body(g, carry):
                tot = jnp.zeros((L,), jnp.float32)
                for r in range(L):
                    row = g * L + r
                    acc = jnp.zeros((L,), jnp.float32)
                    for j in range(d_vecs):
                        hv = hrows[p, row, pl.ds(j * L, L)]
                        tv = trows[p, row, pl.ds(j * L, L)]
                        mv = mrows[p, row, pl.ds(j * L, L)]
                        d = (hv + mv) - tv
                        acc = acc + d * d
                    for sh in (8, 4, 2, 1):
                        acc = acc + _shuffle(acc, iot ^ sh)
                    tot = jnp.where(iot == r, acc, tot)
                outv[pl.ds(c * CHUNK + g * L, L)] = tot
                return carry

            lax.fori_loop(0, CHUNK // L, group_body, 0)

        handles = fire(0)
        for c in range(n_chunks):
            nxt = fire(c + 1) if c + 1 < n_chunks else None
            for hnd in handles:
                hnd.wait()
            compute(c)
            handles = nxt

        pltpu.sync_copy(outv, out_hbm.at[pl.ds(base, b_per_w)])

    return triplet_kernel


def _finish_kernel(B):
    def body(ss_ref, out_ref):
        out_ref[...] = -jnp.sqrt(ss_ref[...])

    return pl.pallas_call(
        body, out_shape=jax.ShapeDtypeStruct((B,), jnp.float32))


def kernel(mention, h, t, emb_table):
    B, D = mention.shape
    V = emb_table.shape[0]
    ss = _make_kernel(B, D, V)(mention, h, t, emb_table)
    return _finish_kernel(B)(ss)


# speedup vs baseline: 1.5519x; 1.2902x over previous
import functools

import jax
import jax.numpy as jnp
from jax import lax
from jax.experimental import pallas as pl
from jax.experimental.pallas import tpu as pltpu
from jax.experimental.pallas import tpu_sc as plsc

_GATHER_DNUMS = lax.GatherDimensionNumbers(
    offset_dims=(), collapsed_slice_dims=(0,), start_index_map=(0,))


def _shuffle(v, perm):
    return lax.gather(v, perm[:, None], _GATHER_DNUMS, slice_sizes=(1,),
                      mode=lax.GatherScatterMode.PROMISE_IN_BOUNDS)


L = 16
NC = 2
NS = 16
NW = NC * NS
CHUNK = 128


def _make_kernel(B, D, V):
    n_chunks = B // (NW * CHUNK)
    b_per_w = n_chunks * CHUNK
    d_vecs = D // L

    mesh = plsc.VectorSubcoreMesh(core_axis_name="c", subcore_axis_name="s")

    @functools.partial(
        pl.kernel,
        mesh=mesh,
        out_type=jax.ShapeDtypeStruct((B,), jnp.float32),
        scratch_types=[
            pltpu.VMEM((b_per_w,), jnp.int32),
            pltpu.VMEM((b_per_w,), jnp.int32),
            pltpu.VMEM((2, CHUNK, D), jnp.float32),
            pltpu.VMEM((2, CHUNK, D), jnp.float32),
            pltpu.VMEM((2, CHUNK, D), jnp.float32),
            pltpu.VMEM((b_per_w,), jnp.float32),
            pltpu.SemaphoreType.DMA,
            pltpu.SemaphoreType.DMA,
            pltpu.SemaphoreType.DMA,
            pltpu.SemaphoreType.DMA,
            pltpu.SemaphoreType.DMA,
            pltpu.SemaphoreType.DMA,
        ],
    )
    def triplet_kernel(mention_hbm, h_hbm, t_hbm, table_hbm, out_hbm,
                       hidx, tidx, hrows, trows, mrows, outv,
                       hs0, hs1, ts0, ts1, ms0, ms1):
        wid = lax.axis_index("s") * NC + lax.axis_index("c")
        iot = lax.iota(jnp.int32, L)
        base = wid * b_per_w
        hsem = (hs0, hs1)
        tsem = (ts0, ts1)
        msem = (ms0, ms1)

        pltpu.sync_copy(h_hbm.at[pl.ds(base, b_per_w)], hidx)
        pltpu.sync_copy(t_hbm.at[pl.ds(base, b_per_w)], tidx)

        def fire(c, p):
            return (
                pltpu.async_copy(
                    table_hbm.at[hidx.at[pl.ds(c * CHUNK, CHUNK)]],
                    hrows.at[p], hsem[p]),
                pltpu.async_copy(
                    table_hbm.at[tidx.at[pl.ds(c * CHUNK, CHUNK)]],
                    trows.at[p], tsem[p]),
                pltpu.async_copy(
                    mention_hbm.at[pl.ds(base + c * CHUNK, CHUNK)],
                    mrows.at[p], msem[p]),
            )

        def wait(c, p):
            pltpu.make_async_copy(
                table_hbm.at[hidx.at[pl.ds(c * CHUNK, CHUNK)]],
                hrows.at[p], hsem[p]).wait()
            pltpu.make_async_copy(
                table_hbm.at[tidx.at[pl.ds(c * CHUNK, CHUNK)]],
                trows.at[p], tsem[p]).wait()
            pltpu.make_async_copy(
                mention_hbm.at[pl.ds(base + c * CHUNK, CHUNK)],
                mrows.at[p], msem[p]).wait()

        def compute(c, p):
            def group_body(g, carry):
                def sub_body(k, tot):
                    for rr in range(4):
                        r = k * 4 + rr
                        row = g * L + r
                        acc = jnp.zeros((L,), jnp.float32)
                        for j in range(d_vecs):
                            hv = hrows[p, row, pl.ds(j * L, L)]
                            tv = trows[p, row, pl.ds(j * L, L)]
                            mv = mrows[p, row, pl.ds(j * L, L)]
                            d = (hv + mv) - tv
                            acc = acc + d * d
                        for sh in (8, 4, 2, 1):
                            acc = acc + _shuffle(acc, iot ^ sh)
                        tot = jnp.where(iot == r, acc, tot)
                    return tot

                tot = lax.fori_loop(0, 4, sub_body,
                                    jnp.zeros((L,), jnp.float32))
                outv[pl.ds(c * CHUNK + g * L, L)] = tot
                return carry

            lax.fori_loop(0, CHUNK // L, group_body, 0)

        fire(0, 0)

        def pair_body(i, carry):
            c0 = 2 * i
            fire(c0 + 1, 1)
            wait(c0, 0)
            compute(c0, 0)

            @pl.when(c0 + 2 < n_chunks)
            def _():
                fire(c0 + 2, 0)

            wait(c0 + 1, 1)
            compute(c0 + 1, 1)
            return carry

        lax.fori_loop(0, n_chunks // 2, pair_body, 0)

        pltpu.sync_copy(outv, out_hbm.at[pl.ds(base, b_per_w)])

    return triplet_kernel


def _finish_kernel(B):
    def body(ss_ref, out_ref):
        out_ref[...] = -jnp.sqrt(ss_ref[...])

    return pl.pallas_call(
        body, out_shape=jax.ShapeDtypeStruct((B,), jnp.float32))


def kernel(mention, h, t, emb_table):
    B, D = mention.shape
    V = emb_table.shape[0]
    ss = _make_kernel(B, D, V)(mention, h, t, emb_table)
    return _finish_kernel(B)(ss)


# speedup vs baseline: 1.5688x; 1.0109x over previous
import functools

import jax
import jax.numpy as jnp
from jax import lax
from jax.experimental import pallas as pl
from jax.experimental.pallas import tpu as pltpu
from jax.experimental.pallas import tpu_sc as plsc

_GATHER_DNUMS = lax.GatherDimensionNumbers(
    offset_dims=(), collapsed_slice_dims=(0,), start_index_map=(0,))


def _shuffle(v, perm):
    return lax.gather(v, perm[:, None], _GATHER_DNUMS, slice_sizes=(1,),
                      mode=lax.GatherScatterMode.PROMISE_IN_BOUNDS)


L = 16
NC = 2
NS = 16
NW = NC * NS
CHUNK = 128


def _make_kernel(B, D, V):
    n_chunks = B // (NW * CHUNK)
    b_per_w = n_chunks * CHUNK
    d_vecs = D // L

    mesh = plsc.VectorSubcoreMesh(core_axis_name="c", subcore_axis_name="s")

    @functools.partial(
        pl.kernel,
        mesh=mesh,
        out_type=jax.ShapeDtypeStruct((B,), jnp.float32),
        scratch_types=[
            pltpu.VMEM((b_per_w,), jnp.int32),
            pltpu.VMEM((b_per_w,), jnp.int32),
            pltpu.VMEM((2, CHUNK, D), jnp.float32),
            pltpu.VMEM((2, CHUNK, D), jnp.float32),
            pltpu.VMEM((2, CHUNK, D), jnp.float32),
            pltpu.VMEM((b_per_w,), jnp.float32),
            pltpu.SemaphoreType.DMA,
            pltpu.SemaphoreType.DMA,
            pltpu.SemaphoreType.DMA,
            pltpu.SemaphoreType.DMA,
            pltpu.SemaphoreType.DMA,
            pltpu.SemaphoreType.DMA,
        ],
    )
    def triplet_kernel(mention_hbm, h_hbm, t_hbm, table_hbm, out_hbm,
                       hidx, tidx, hrows, trows, mrows, outv,
                       hs0, hs1, ts0, ts1, ms0, ms1):
        wid = lax.axis_index("s") * NC + lax.axis_index("c")
        iot = lax.iota(jnp.int32, L)
        base = wid * b_per_w
        hsem = (hs0, hs1)
        tsem = (ts0, ts1)
        msem = (ms0, ms1)

        HALF = CHUNK // 2

        def fire_mention(c, p):
            return pltpu.async_copy(
                mention_hbm.at[pl.ds(base + c * CHUNK, CHUNK)],
                mrows.at[p], msem[p])

        def fire_gathers(c, p):
            for half in range(2):
                pltpu.async_copy(
                    table_hbm.at[hidx.at[pl.ds(c * CHUNK + half * HALF,
                                               HALF)]],
                    hrows.at[p].at[pl.ds(half * HALF, HALF)], hsem[p])
                pltpu.async_copy(
                    table_hbm.at[tidx.at[pl.ds(c * CHUNK + half * HALF,
                                               HALF)]],
                    trows.at[p].at[pl.ds(half * HALF, HALF)], tsem[p])

        def fire(c, p):
            fire_mention(c, p)
            fire_gathers(c, p)

        def wait(c, p):
            pltpu.make_async_copy(
                table_hbm.at[hidx.at[pl.ds(c * CHUNK, CHUNK)]],
                hrows.at[p], hsem[p]).wait()
            pltpu.make_async_copy(
                table_hbm.at[tidx.at[pl.ds(c * CHUNK, CHUNK)]],
                trows.at[p], tsem[p]).wait()
            pltpu.make_async_copy(
                mention_hbm.at[pl.ds(base + c * CHUNK, CHUNK)],
                mrows.at[p], msem[p]).wait()

        mcp0 = fire_mention(0, 0)
        hicp = pltpu.async_copy(h_hbm.at[pl.ds(base, b_per_w)], hidx, hs1)
        ticp = pltpu.async_copy(t_hbm.at[pl.ds(base, b_per_w)], tidx, ts1)
        hicp.wait()
        ticp.wait()

        def compute(c, p):
            def group_body(g, carry):
                def sub_body(k, tot):
                    for rr in range(4):
                        r = k * 4 + rr
                        row = g * L + r
                        acc = jnp.zeros((L,), jnp.float32)
                        for j in range(d_vecs):
                            hv = hrows[p, row, pl.ds(j * L, L)]
                            tv = trows[p, row, pl.ds(j * L, L)]
                            mv = mrows[p, row, pl.ds(j * L, L)]
                            d = (hv + mv) - tv
                            acc = acc + d * d
                        for sh in (8, 4, 2, 1):
                            acc = acc + _shuffle(acc, iot ^ sh)
                        tot = jnp.where(iot == r, acc, tot)
                    return tot

                tot = lax.fori_loop(0, 4, sub_body,
                                    jnp.zeros((L,), jnp.float32))
                outv[pl.ds(c * CHUNK + g * L, L)] = tot
                return carry

            lax.fori_loop(0, CHUNK // L, group_body, 0)

        fire_gathers(0, 0)

        def pair_body(i, carry):
            c0 = 2 * i
            fire(c0 + 1, 1)
            wait(c0, 0)
            compute(c0, 0)

            @pl.when(c0 + 2 < n_chunks)
            def _():
                fire(c0 + 2, 0)

            wait(c0 + 1, 1)
            compute(c0 + 1, 1)
            return carry

        lax.fori_loop(0, n_chunks // 2, pair_body, 0)

        pltpu.sync_copy(outv, out_hbm.at[pl.ds(base, b_per_w)])

    return triplet_kernel


def _finish_kernel(B):
    def body(ss_ref, out_ref):
        out_ref[...] = -jnp.sqrt(ss_ref[...])

    return pl.pallas_call(
        body, out_shape=jax.ShapeDtypeStruct((B,), jnp.float32))


def kernel(mention, h, t, emb_table):
    B, D = mention.shape
    V = emb_table.shape[0]
    ss = _make_kernel(B, D, V)(mention, h, t, emb_table)
    return _finish_kernel(B)(ss)


# speedup vs baseline: 1.6604x; 1.0584x over previous
import functools

import jax
import jax.numpy as jnp
from jax import lax
from jax.experimental import pallas as pl
from jax.experimental.pallas import tpu as pltpu
from jax.experimental.pallas import tpu_sc as plsc

_GATHER_DNUMS = lax.GatherDimensionNumbers(
    offset_dims=(), collapsed_slice_dims=(0,), start_index_map=(0,))


def _shuffle(v, perm):
    return lax.gather(v, perm[:, None], _GATHER_DNUMS, slice_sizes=(1,),
                      mode=lax.GatherScatterMode.PROMISE_IN_BOUNDS)


L = 16
NC = 2
NS = 16
NW = NC * NS
CHUNK = 128


def _make_kernel(B, D, V):
    n_chunks = B // (NW * CHUNK)
    b_per_w = n_chunks * CHUNK
    d_vecs = D // L

    mesh = plsc.VectorSubcoreMesh(core_axis_name="c", subcore_axis_name="s")

    @functools.partial(
        pl.kernel,
        mesh=mesh,
        out_type=jax.ShapeDtypeStruct((B,), jnp.float32),
        scratch_types=[
            pltpu.VMEM((b_per_w,), jnp.int32),
            pltpu.VMEM((b_per_w,), jnp.int32),
            pltpu.VMEM((2, CHUNK, D), jnp.float32),
            pltpu.VMEM((2, CHUNK, D), jnp.float32),
            pltpu.VMEM((2, CHUNK, D), jnp.float32),
            pltpu.VMEM((b_per_w,), jnp.float32),
            pltpu.SemaphoreType.DMA,
            pltpu.SemaphoreType.DMA,
            pltpu.SemaphoreType.DMA,
            pltpu.SemaphoreType.DMA,
            pltpu.SemaphoreType.DMA,
            pltpu.SemaphoreType.DMA,
        ],
    )
    def triplet_kernel(mention_hbm, h_hbm, t_hbm, table_hbm, out_hbm,
                       hidx, tidx, hrows, trows, mrows, outv,
                       hs0, hs1, ts0, ts1, ms0, ms1):
        wid = lax.axis_index("s") * NC + lax.axis_index("c")
        iot = lax.iota(jnp.int32, L)
        base = wid * b_per_w
        hsem = (hs0, hs1)
        tsem = (ts0, ts1)
        msem = (ms0, ms1)

        HALF = CHUNK // 2

        def fire_mention(c, p):
            return pltpu.async_copy(
                mention_hbm.at[pl.ds(base + c * CHUNK, CHUNK)],
                mrows.at[p], msem[p])

        def fire_gathers(c, p):
            for half in range(2):
                pltpu.async_copy(
                    table_hbm.at[hidx.at[pl.ds(c * CHUNK + half * HALF,
                                               HALF)]],
                    hrows.at[p].at[pl.ds(half * HALF, HALF)], hsem[p])
                pltpu.async_copy(
                    table_hbm.at[tidx.at[pl.ds(c * CHUNK + half * HALF,
                                               HALF)]],
                    trows.at[p].at[pl.ds(half * HALF, HALF)], tsem[p])

        def fire(c, p):
            fire_mention(c, p)
            fire_gathers(c, p)

        def wait(c, p):
            pltpu.make_async_copy(
                table_hbm.at[hidx.at[pl.ds(c * CHUNK, CHUNK)]],
                hrows.at[p], hsem[p]).wait()
            pltpu.make_async_copy(
                table_hbm.at[tidx.at[pl.ds(c * CHUNK, CHUNK)]],
                trows.at[p], tsem[p]).wait()
            pltpu.make_async_copy(
                mention_hbm.at[pl.ds(base + c * CHUNK, CHUNK)],
                mrows.at[p], msem[p]).wait()

        mcp0 = fire_mention(0, 0)
        hicp = pltpu.async_copy(h_hbm.at[pl.ds(base, b_per_w)], hidx, hs1)
        ticp = pltpu.async_copy(t_hbm.at[pl.ds(base, b_per_w)], tidx, ts1)
        hicp.wait()
        ticp.wait()

        def compute(c, p):
            def group_body(g, carry):
                def sub_body(k, tot):
                    for rr in range(2):
                        r = k * 2 + rr
                        row = g * L + r
                        acc = jnp.zeros((L,), jnp.float32)
                        for j in range(d_vecs):
                            hv = hrows[p, row, pl.ds(j * L, L)]
                            tv = trows[p, row, pl.ds(j * L, L)]
                            mv = mrows[p, row, pl.ds(j * L, L)]
                            d = (hv + mv) - tv
                            acc = acc + d * d
                        for sh in (8, 4, 2, 1):
                            acc = acc + _shuffle(acc, iot ^ sh)
                        tot = jnp.where(iot == r, acc, tot)
                    return tot

                tot = lax.fori_loop(0, 8, sub_body,
                                    jnp.zeros((L,), jnp.float32))
                outv[pl.ds(c * CHUNK + g * L, L)] = tot
                return carry

            lax.fori_loop(0, CHUNK // L, group_body, 0)

        fire_gathers(0, 0)

        def pair_body(i, carry):
            c0 = 2 * i
            fire(c0 + 1, 1)
            wait(c0, 0)
            compute(c0, 0)

            @pl.when(c0 + 2 < n_chunks)
            def _():
                fire(c0 + 2, 0)

            wait(c0 + 1, 1)
            compute(c0 + 1, 1)
            return carry

        lax.fori_loop(0, n_chunks // 2, pair_body, 0)

        pltpu.sync_copy(outv, out_hbm.at[pl.ds(base, b_per_w)])

    return triplet_kernel


def _finish_kernel(B):
    def body(ss_ref, out_ref):
        out_ref[...] = -jnp.sqrt(ss_ref[...])

    return pl.pallas_call(
        body, out_shape=jax.ShapeDtypeStruct((B,), jnp.float32))


def kernel(mention, h, t, emb_table):
    B, D = mention.shape
    V = emb_table.shape[0]
    ss = _make_kernel(B, D, V)(mention, h, t, emb_table)
    return _finish_kernel(B)(ss)
